# Initial kernel scaffold; baseline (speedup 1.0000x reference)
import jax
import jax.numpy as jnp
from jax.experimental import pallas as pl


def kernel(x, prototype_labels):
    raise NotImplementedError("write your pallas kernel here")


# speedup vs baseline: 18.4843x; 18.4843x over previous
import jax
import jax.numpy as jnp
from jax.experimental import pallas as pl

B = 16384
N = 256
BLOCK_ROWS = 1024
NUM_BLOCKS = B // BLOCK_ROWS


def _wtac_block(x_ref, lab_ref, out_ref):
    x = x_ref[...]
    col = jax.lax.broadcasted_iota(jnp.int32, (BLOCK_ROWS, N), 1)
    m = jnp.min(x, axis=1, keepdims=True)
    idxm = jnp.where(x == m, col, jnp.int32(2**30))
    win = jnp.min(idxm, axis=1, keepdims=True)
    sel = idxm == win
    lab = lab_ref[...]
    y = jnp.sum(jnp.where(sel, lab, 0.0), axis=1)
    out_ref[...] = y


def kernel(x, prototype_labels):
    lab2d = prototype_labels.reshape(1, N)
    out = pl.pallas_call(
        _wtac_block,
        grid=(NUM_BLOCKS,),
        in_specs=[
            pl.BlockSpec((BLOCK_ROWS, N), lambda i: (i, 0)),
            pl.BlockSpec((1, N), lambda i: (0, 0)),
        ],
        out_specs=pl.BlockSpec((BLOCK_ROWS,), lambda i: (i,)),
        out_shape=jax.ShapeDtypeStruct((B,), jnp.float32),
    )(x, lab2d)
    return out
